# baseline (speedup 1.0000x reference)
import jax
import jax.numpy as jnp
from jax.experimental import pallas as pl
from jax.experimental.pallas import tpu as pltpu

B = 64
P = 1024
MAX_PATCH = 32
EMB = 64
PATCH_DIM = 256
FLAT = EMB * MAX_PATCH


def _body(ids_ref, table_ref, W1_ref, b1_ref, W2_ref, b2_ref, gamma_ref,
          beta_ref, out_ref, bigT_ref, ohA_ref, ohB_ref):
    b = pl.program_id(0)

    @pl.when(b == 0)
    def _():
        tab = table_ref[...]
        for j in range(MAX_PATCH):
            w1j = W1_ref[pl.ds(j * EMB, EMB), :]
            bigT_ref[pl.ds(j * 256, 256), :] = jnp.dot(
                tab, w1j, preferred_element_type=jnp.float32
            ).astype(jnp.bfloat16)

    parity = b % 2

    def build(oh_ref):
        ids16 = ids_ref[0].astype(jnp.int16)
        iota16 = jax.lax.broadcasted_iota(jnp.int16, (P, PATCH_DIM), 1)
        for j in range(MAX_PATCH):
            col = ids16[:, j:j + 1]
            oh = jnp.where(col == iota16, jnp.bfloat16(1), jnp.bfloat16(0))
            oh_ref[:, pl.ds(j * 256, 256)] = oh

    def consume(oh_ref):
        h = jnp.dot(oh_ref[...], bigT_ref[...],
                    preferred_element_type=jnp.float32)
        h = h + b1_ref[0]
        h = h * 0.5 * (1.0 + jax.lax.erf(h * 0.7071067811865476))
        h = jnp.dot(h.astype(jnp.bfloat16),
                    W2_ref[...].astype(jnp.bfloat16),
                    preferred_element_type=jnp.float32) + b2_ref[0]

        mu = jnp.mean(h, axis=1, keepdims=True)
        var = jnp.mean(jnp.square(h - mu), axis=1, keepdims=True)
        y = (h - mu) * jax.lax.rsqrt(var + 1e-5)
        y = y * gamma_ref[0] + beta_ref[0]
        out_ref[0, 0, :] = jnp.mean(y, axis=0)

    @pl.when((b < B) & (parity == 0))
    def _():
        build(ohA_ref)

    @pl.when((b < B) & (parity == 1))
    def _():
        build(ohB_ref)

    @pl.when((b > 0) & (parity == 1))
    def _():
        consume(ohA_ref)

    @pl.when((b > 0) & (parity == 0))
    def _():
        consume(ohB_ref)


def kernel(byte_ids, table, W1, b1, W2, b2, gamma, beta):
    ids3 = byte_ids.reshape(B, P, MAX_PATCH)
    b1r = b1.reshape(1, PATCH_DIM)
    b2r = b2.reshape(1, PATCH_DIM)
    gammar = gamma.reshape(1, PATCH_DIM)
    betar = beta.reshape(1, PATCH_DIM)

    grid = (B + 1,)
    out = pl.pallas_call(
        _body,
        grid=grid,
        in_specs=[
            pl.BlockSpec((1, P, MAX_PATCH),
                         lambda b: (jnp.minimum(b, B - 1), 0, 0)),
            pl.BlockSpec((256, EMB), lambda b: (0, 0)),
            pl.BlockSpec((FLAT, PATCH_DIM), lambda b: (0, 0)),
            pl.BlockSpec((1, PATCH_DIM), lambda b: (0, 0)),
            pl.BlockSpec((PATCH_DIM, PATCH_DIM), lambda b: (0, 0)),
            pl.BlockSpec((1, PATCH_DIM), lambda b: (0, 0)),
            pl.BlockSpec((1, PATCH_DIM), lambda b: (0, 0)),
            pl.BlockSpec((1, PATCH_DIM), lambda b: (0, 0)),
        ],
        out_specs=pl.BlockSpec((1, 1, PATCH_DIM),
                               lambda b: (jnp.maximum(b - 1, 0), 0, 0)),
        out_shape=jax.ShapeDtypeStruct((B, 1, PATCH_DIM), jnp.float32),
        scratch_shapes=[
            pltpu.VMEM((MAX_PATCH * 256, PATCH_DIM), jnp.bfloat16),
            pltpu.VMEM((P, MAX_PATCH * 256), jnp.bfloat16),
            pltpu.VMEM((P, MAX_PATCH * 256), jnp.bfloat16),
        ],
        compiler_params=pltpu.CompilerParams(
            dimension_semantics=("arbitrary",),
            vmem_limit_bytes=100 * 1024 * 1024,
        ),
    )(ids3, table, W1, b1r, W2, b2r, gammar, betar)
    return out.reshape(B, PATCH_DIM)


# speedup vs baseline: 1.4377x; 1.4377x over previous
import jax
import jax.numpy as jnp
from jax.experimental import pallas as pl
from jax.experimental.pallas import tpu as pltpu

B = 64
P = 1024
MAX_PATCH = 32
EMB = 64
PATCH_DIM = 256
FLAT = EMB * MAX_PATCH


def _body(ids_ref, table_ref, W1_ref, b1_ref, W2_ref, b2_ref, gamma_ref,
          beta_ref, out_ref, bigT_ref, oh_ref):
    @pl.when(pl.program_id(0) == 0)
    def _():
        tab = table_ref[...]
        for j in range(MAX_PATCH):
            w1j = W1_ref[pl.ds(j * EMB, EMB), :]
            bigT_ref[pl.ds(j * 256, 256), :] = jnp.dot(
                tab, w1j, preferred_element_type=jnp.float32
            ).astype(jnp.bfloat16)

    ids16 = ids_ref[0].astype(jnp.int16)
    iota16 = jax.lax.broadcasted_iota(jnp.int16, (P, PATCH_DIM), 1)

    for j in range(MAX_PATCH):
        col = ids16[:, j:j + 1]
        oh = jnp.where(col == iota16, jnp.bfloat16(1), jnp.bfloat16(0))
        oh_ref[:, pl.ds(j * 256, 256)] = oh

    h = jnp.dot(oh_ref[...], bigT_ref[...],
                preferred_element_type=jnp.float32)

    h = h + b1_ref[0]
    h = h * 0.5 * (1.0 + jax.lax.erf(h * 0.7071067811865476))
    h = jnp.dot(h.astype(jnp.bfloat16), W2_ref[...].astype(jnp.bfloat16),
                preferred_element_type=jnp.float32) + b2_ref[0]

    mu = jnp.mean(h, axis=1, keepdims=True)
    var = jnp.mean(jnp.square(h - mu), axis=1, keepdims=True)
    y = (h - mu) * jax.lax.rsqrt(var + 1e-5)
    y = y * gamma_ref[0] + beta_ref[0]

    out_ref[0, 0, :] = jnp.mean(y, axis=0)


def kernel(byte_ids, table, W1, b1, W2, b2, gamma, beta):
    ids3 = byte_ids.reshape(B, P, MAX_PATCH)
    b1r = b1.reshape(1, PATCH_DIM)
    b2r = b2.reshape(1, PATCH_DIM)
    gammar = gamma.reshape(1, PATCH_DIM)
    betar = beta.reshape(1, PATCH_DIM)

    grid = (B,)
    out = pl.pallas_call(
        _body,
        grid=grid,
        in_specs=[
            pl.BlockSpec((1, P, MAX_PATCH), lambda b: (b, 0, 0)),
            pl.BlockSpec((256, EMB), lambda b: (0, 0)),
            pl.BlockSpec((FLAT, PATCH_DIM), lambda b: (0, 0)),
            pl.BlockSpec((1, PATCH_DIM), lambda b: (0, 0)),
            pl.BlockSpec((PATCH_DIM, PATCH_DIM), lambda b: (0, 0)),
            pl.BlockSpec((1, PATCH_DIM), lambda b: (0, 0)),
            pl.BlockSpec((1, PATCH_DIM), lambda b: (0, 0)),
            pl.BlockSpec((1, PATCH_DIM), lambda b: (0, 0)),
        ],
        out_specs=pl.BlockSpec((1, 1, PATCH_DIM), lambda b: (b, 0, 0)),
        out_shape=jax.ShapeDtypeStruct((B, 1, PATCH_DIM), jnp.float32),
        scratch_shapes=[
            pltpu.VMEM((MAX_PATCH * 256, PATCH_DIM), jnp.bfloat16),
            pltpu.VMEM((P, MAX_PATCH * 256), jnp.bfloat16),
        ],
        compiler_params=pltpu.CompilerParams(
            dimension_semantics=("arbitrary",),
        ),
    )(ids3, table, W1, b1r, W2, b2r, gammar, betar)
    return out.reshape(B, PATCH_DIM)
